# baseline (speedup 1.0000x reference)
import functools

import jax
import jax.numpy as jnp
from jax import lax
from jax.experimental import pallas as pl
from jax.experimental.pallas import tpu as pltpu
from jax.experimental.pallas import tpu_sc as plsc

_ROWS = 8192
_COLS = 4096
_L = 16
_NV = _COLS // _L
_BLK = 16
_NBLK = _NV // _BLK
_CHUNK = 8
_NSLOT = 3
_N_WORKERS = 32
_N_BISECT = 16
_N_NEWTON = 2

_GATHER_DNUMS = lax.GatherDimensionNumbers(
    offset_dims=(), collapsed_slice_dims=(0,), start_index_map=(0,)
)


def _shuffle(v, idx):
    return lax.gather(v, idx[:, None], _GATHER_DNUMS, (1,),
                      mode=lax.GatherScatterMode.PROMISE_IN_BOUNDS)


def _butterfly(v, op):
    iota = lax.iota(jnp.int32, _L)
    for k in (8, 4, 2, 1):
        v = op(v, _shuffle(v, jnp.bitwise_xor(iota, k)))
    return v


def _tree(vals, op):
    vals = list(vals)
    while len(vals) > 1:
        nxt = [op(vals[i], vals[i + 1]) for i in range(0, len(vals) - 1, 2)]
        if len(vals) % 2:
            nxt.append(vals[-1])
        vals = nxt
    return vals[0]


def _sc_body(x_hbm, o_hbm, buf, cand, blks, in_sem, out_sem):
    wid = lax.axis_index("s") * 2 + lax.axis_index("c")
    rows_per_w = _ROWS // _N_WORKERS
    n_chunks = rows_per_w // _CHUNK
    row_base = wid * rows_per_w
    ones = jnp.full((_L,), 1.0, jnp.float32)
    zeros = jnp.zeros((_L,), jnp.float32)

    def in_copy(ci, s):
        return pltpu.make_async_copy(
            x_hbm.at[pl.ds(row_base + ci * _CHUNK, _CHUNK)], buf.at[s],
            in_sem.at[s])

    def out_copy(ci, s):
        return pltpu.make_async_copy(
            buf.at[s], o_hbm.at[pl.ds(row_base + ci * _CHUNK, _CHUNK)],
            out_sem.at[s])

    in_copy(0, 0).start()

    def do_chunk(ci, _):
        s = lax.rem(ci, _NSLOT)
        s_next = lax.rem(ci + 1, _NSLOT)
        in_copy(ci, s).wait()

        @pl.when(jnp.logical_and(ci >= 2, ci + 1 < n_chunks))
        def _():
            out_copy(ci - 2, s_next).wait()

        @pl.when(ci + 1 < n_chunks)
        def _():
            in_copy(ci + 1, s_next).start()

        def ld(r, j):
            return buf[s, r, pl.ds(j * _L, _L)]

        def do_row(r, _):
            def blk_body(b, g):
                def inner(i, accs):
                    return tuple(
                        jnp.maximum(a, ld(r, b * _BLK + i * 4 + k))
                        for k, a in enumerate(accs)
                    )

                a0 = tuple(ld(r, b * _BLK + k) for k in range(4))
                accs = plsc.parallel_loop(1, _BLK // 4, carry=a0)(inner)
                bm = jnp.maximum(jnp.maximum(accs[0], accs[1]),
                                 jnp.maximum(accs[2], accs[3]))
                blks[pl.ds(b * _L, _L)] = bm
                return jnp.maximum(g, bm)

            g = lax.fori_loop(0, _NBLK, blk_body, ld(r, 0))
            m = _butterfly(g, jnp.maximum)
            thr = m - ones

            cnt = jnp.int32(0)
            for b in range(_NBLK):
                has = plsc.all_reduce_population_count(
                    blks[pl.ds(b * _L, _L)] > thr)[0]

                def scan_block(c, b=b):
                    def sb(j, c):
                        v = ld(r, b * _BLK + j)
                        msk = v > thr
                        plsc.store_compressed(
                            cand.at[pl.ds(c, _L)], v, mask=msk)
                        return c + plsc.all_reduce_population_count(msk)[0]

                    return lax.fori_loop(0, _BLK, sb, c)

                cnt = lax.cond(has > 0, scan_block, lambda c: c, cnt)

            cand[pl.ds(cnt, _L)] = thr - ones
            nvc = (cnt + _L - 1) // _L

            def bis(i, lohi):
                lo, hi = lohi
                mid = 0.5 * (lo + hi)

                def inner(k, a):
                    v = cand[pl.ds(k * _L, _L)]
                    return a + jnp.maximum(v - mid, 0.0)

                a = lax.fori_loop(0, nvc, inner, zeros)
                p = _butterfly(a, jnp.add) >= ones
                return jnp.where(p, mid, lo), jnp.where(p, hi, mid)

            lo, hi = lax.fori_loop(0, _N_BISECT, bis, (thr, m))
            tau0 = 0.5 * (lo + hi)

            def newton(i, tau):
                def inner(k, carry):
                    sa, ca = carry
                    v = cand[pl.ds(k * _L, _L)]
                    d = v - tau
                    sa = sa + jnp.maximum(d, 0.0)
                    ca = ca + jnp.where(d > zeros, 1.0, 0.0)
                    return sa, ca

                sa, ca = lax.fori_loop(0, nvc, inner, (zeros, zeros))
                s_ = _butterfly(sa, jnp.add)
                c_ = _butterfly(ca, jnp.add)
                return tau + (s_ - ones) / jnp.maximum(c_, ones)

            tau = lax.fori_loop(0, _N_NEWTON, newton, tau0)

            def out_body(i):
                for k in range(8):
                    sl = pl.ds((i * 8 + k) * _L, _L)
                    buf[s, r, sl] = jnp.maximum(buf[s, r, sl] - tau, 0.0)

            plsc.parallel_loop(0, _NV // 8)(out_body)
            return 0

        lax.fori_loop(0, _CHUNK, do_row, 0)
        out_copy(ci, s).start()
        return 0

    lax.fori_loop(0, n_chunks, do_chunk, 0)
    for ci in (n_chunks - 3, n_chunks - 2, n_chunks - 1):
        out_copy(ci, ci % _NSLOT).wait()


def _sparsemax_sc(x):
    mesh = plsc.VectorSubcoreMesh(core_axis_name="c", subcore_axis_name="s")
    f = pl.kernel(
        _sc_body,
        out_type=jax.ShapeDtypeStruct((_ROWS, _COLS), jnp.float32),
        mesh=mesh,
        scratch_types=[
            pltpu.VMEM((_NSLOT, _CHUNK, _COLS), jnp.float32),
            pltpu.VMEM((_COLS + _L,), jnp.float32),
            pltpu.VMEM((_NBLK * _L,), jnp.float32),
            pltpu.SemaphoreType.DMA((_NSLOT,)),
            pltpu.SemaphoreType.DMA((_NSLOT,)),
        ],
        compiler_params=pltpu.CompilerParams(needs_layout_passes=False),
    )
    return f(x)


def kernel(input):
    return _sparsemax_sc(input)


# speedup vs baseline: 1.0007x; 1.0007x over previous
import functools

import jax
import jax.numpy as jnp
from jax import lax
from jax.experimental import pallas as pl
from jax.experimental.pallas import tpu as pltpu
from jax.experimental.pallas import tpu_sc as plsc

_ROWS = 8192
_COLS = 4096
_L = 16
_NV = _COLS // _L
_BLK = 16
_NBLK = _NV // _BLK
_CHUNK = 8
_NSLOT = 3
_N_WORKERS = 32
_N_BISECT = 16
_N_NEWTON = 2

_GATHER_DNUMS = lax.GatherDimensionNumbers(
    offset_dims=(), collapsed_slice_dims=(0,), start_index_map=(0,)
)


def _shuffle(v, idx):
    return lax.gather(v, idx[:, None], _GATHER_DNUMS, (1,),
                      mode=lax.GatherScatterMode.PROMISE_IN_BOUNDS)


def _butterfly(v, op):
    iota = lax.iota(jnp.int32, _L)
    for k in (8, 4, 2, 1):
        v = op(v, _shuffle(v, jnp.bitwise_xor(iota, k)))
    return v


def _tree(vals, op):
    vals = list(vals)
    while len(vals) > 1:
        nxt = [op(vals[i], vals[i + 1]) for i in range(0, len(vals) - 1, 2)]
        if len(vals) % 2:
            nxt.append(vals[-1])
        vals = nxt
    return vals[0]


def _sc_body(x_hbm, o_hbm, buf, cand, blks, in_sem, out_sem):
    wid = lax.axis_index("s") * 2 + lax.axis_index("c")
    rows_per_w = _ROWS // _N_WORKERS
    n_chunks = rows_per_w // _CHUNK
    row_base = wid * rows_per_w
    ones = jnp.full((_L,), 1.0, jnp.float32)
    zeros = jnp.zeros((_L,), jnp.float32)

    def in_copy(ci, s):
        return pltpu.make_async_copy(
            x_hbm.at[pl.ds(row_base + ci * _CHUNK, _CHUNK)], buf.at[s],
            in_sem.at[s])

    def out_copy(ci, s):
        return pltpu.make_async_copy(
            buf.at[s], o_hbm.at[pl.ds(row_base + ci * _CHUNK, _CHUNK)],
            out_sem.at[s])

    in_copy(0, 0).start()

    def do_chunk(ci, _):
        s = lax.rem(ci, _NSLOT)
        s_next = lax.rem(ci + 1, _NSLOT)
        in_copy(ci, s).wait()

        @pl.when(jnp.logical_and(ci >= 2, ci + 1 < n_chunks))
        def _():
            out_copy(ci - 2, s_next).wait()

        @pl.when(ci + 1 < n_chunks)
        def _():
            in_copy(ci + 1, s_next).start()

        def ld(r, j):
            return buf[s, r, pl.ds(j * _L, _L)]

        def do_row(r, _):
            def blk_body(b, g):
                def inner(i, accs):
                    return tuple(
                        jnp.maximum(a, ld(r, b * _BLK + i * 4 + k))
                        for k, a in enumerate(accs)
                    )

                a0 = tuple(ld(r, b * _BLK + k) for k in range(4))
                accs = plsc.parallel_loop(1, _BLK // 4, carry=a0)(inner)
                bm = jnp.maximum(jnp.maximum(accs[0], accs[1]),
                                 jnp.maximum(accs[2], accs[3]))
                blks[pl.ds(b * _L, _L)] = bm
                return jnp.maximum(g, bm)

            g = lax.fori_loop(0, _NBLK, blk_body, ld(r, 0))
            m = _butterfly(g, jnp.maximum)
            thr = m - ones

            cnt = jnp.int32(0)
            for b in range(_NBLK):
                has = plsc.all_reduce_population_count(
                    blks[pl.ds(b * _L, _L)] > thr)[0]

                def scan_block(c, b=b):
                    def sb(j, c):
                        v = ld(r, b * _BLK + j)
                        msk = v > thr
                        plsc.store_compressed(
                            cand.at[pl.ds(c, _L)], v, mask=msk)
                        return c + plsc.all_reduce_population_count(msk)[0]

                    return lax.fori_loop(0, _BLK, sb, c)

                cnt = lax.cond(has > 0, scan_block, lambda c: c, cnt)

            cand[pl.ds(cnt, _L)] = thr - ones
            nvc = (cnt + _L - 1) // _L

            def solve_reg(_):
                v0 = cand[pl.ds(0, _L)]
                sp = [jnp.broadcast_to(v0[i], (_L,)) for i in range(_L)]

                def bis(i, lohi):
                    lo, hi = lohi
                    mid = 0.5 * (lo + hi)
                    s_ = _tree([jnp.maximum(v - mid, 0.0) for v in sp], jnp.add)
                    p = s_ >= ones
                    return jnp.where(p, mid, lo), jnp.where(p, hi, mid)

                lo, hi = lax.fori_loop(0, _N_BISECT, bis, (thr, m))
                tau = 0.5 * (lo + hi)

                def newton(i, tau):
                    ds_ = [v - tau for v in sp]
                    s_ = _tree([jnp.maximum(d, 0.0) for d in ds_], jnp.add)
                    c_ = _tree([jnp.where(d > zeros, 1.0, 0.0) for d in ds_],
                               jnp.add)
                    return tau + (s_ - ones) / jnp.maximum(c_, ones)

                return lax.fori_loop(0, _N_NEWTON, newton, tau)

            def solve_loop(_):
                def bis(i, lohi):
                    lo, hi = lohi
                    mid = 0.5 * (lo + hi)

                    def inner(k, a):
                        v = cand[pl.ds(k * _L, _L)]
                        return a + jnp.maximum(v - mid, 0.0)

                    a = lax.fori_loop(0, nvc, inner, zeros)
                    p = _butterfly(a, jnp.add) >= ones
                    return jnp.where(p, mid, lo), jnp.where(p, hi, mid)

                lo, hi = lax.fori_loop(0, _N_BISECT, bis, (thr, m))
                tau = 0.5 * (lo + hi)

                def newton(i, tau):
                    def inner(k, carry):
                        sa, ca = carry
                        v = cand[pl.ds(k * _L, _L)]
                        d = v - tau
                        sa = sa + jnp.maximum(d, 0.0)
                        ca = ca + jnp.where(d > zeros, 1.0, 0.0)
                        return sa, ca

                    sa, ca = lax.fori_loop(0, nvc, inner, (zeros, zeros))
                    s_ = _butterfly(sa, jnp.add)
                    c_ = _butterfly(ca, jnp.add)
                    return tau + (s_ - ones) / jnp.maximum(c_, ones)

                return lax.fori_loop(0, _N_NEWTON, newton, tau)

            tau = lax.cond(cnt <= _L, solve_reg, solve_loop, 0)

            def out_body(i):
                for k in range(8):
                    sl = pl.ds((i * 8 + k) * _L, _L)
                    buf[s, r, sl] = jnp.maximum(buf[s, r, sl] - tau, 0.0)

            plsc.parallel_loop(0, _NV // 8)(out_body)
            return 0

        lax.fori_loop(0, _CHUNK, do_row, 0)
        out_copy(ci, s).start()
        return 0

    lax.fori_loop(0, n_chunks, do_chunk, 0)
    for ci in (n_chunks - 3, n_chunks - 2, n_chunks - 1):
        out_copy(ci, ci % _NSLOT).wait()


def _sparsemax_sc(x):
    mesh = plsc.VectorSubcoreMesh(core_axis_name="c", subcore_axis_name="s")
    f = pl.kernel(
        _sc_body,
        out_type=jax.ShapeDtypeStruct((_ROWS, _COLS), jnp.float32),
        mesh=mesh,
        scratch_types=[
            pltpu.VMEM((_NSLOT, _CHUNK, _COLS), jnp.float32),
            pltpu.VMEM((_COLS + _L,), jnp.float32),
            pltpu.VMEM((_NBLK * _L,), jnp.float32),
            pltpu.SemaphoreType.DMA((_NSLOT,)),
            pltpu.SemaphoreType.DMA((_NSLOT,)),
        ],
        compiler_params=pltpu.CompilerParams(needs_layout_passes=False),
    )
    return f(x)


def kernel(input):
    return _sparsemax_sc(input)


# speedup vs baseline: 1.6615x; 1.6604x over previous
import functools

import jax
import jax.numpy as jnp
from jax import lax
from jax.experimental import pallas as pl
from jax.experimental.pallas import tpu as pltpu
from jax.experimental.pallas import tpu_sc as plsc

_ROWS = 8192
_COLS = 4096
_L = 16
_NV = _COLS // _L
_BLK = 16
_NBLK = _NV // _BLK
_CHUNK = 8
_NSLOT = 3
_N_WORKERS = 32
_N_BISECT = 16
_N_NEWTON = 2

_GATHER_DNUMS = lax.GatherDimensionNumbers(
    offset_dims=(), collapsed_slice_dims=(0,), start_index_map=(0,)
)


def _shuffle(v, idx):
    return lax.gather(v, idx[:, None], _GATHER_DNUMS, (1,),
                      mode=lax.GatherScatterMode.PROMISE_IN_BOUNDS)


def _butterfly(v, op):
    iota = lax.iota(jnp.int32, _L)
    for k in (8, 4, 2, 1):
        v = op(v, _shuffle(v, jnp.bitwise_xor(iota, k)))
    return v


def _tree(vals, op):
    vals = list(vals)
    while len(vals) > 1:
        nxt = [op(vals[i], vals[i + 1]) for i in range(0, len(vals) - 1, 2)]
        if len(vals) % 2:
            nxt.append(vals[-1])
        vals = nxt
    return vals[0]


def _sc_body(x_hbm, o_hbm, buf, cand, blks, in_sem, out_sem):
    wid = lax.axis_index("s") * 2 + lax.axis_index("c")
    rows_per_w = _ROWS // _N_WORKERS
    n_chunks = rows_per_w // _CHUNK
    row_base = wid * rows_per_w
    ones = jnp.full((_L,), 1.0, jnp.float32)
    zeros = jnp.zeros((_L,), jnp.float32)

    def in_copy(ci, s):
        return pltpu.make_async_copy(
            x_hbm.at[pl.ds(row_base + ci * _CHUNK, _CHUNK)], buf.at[s],
            in_sem.at[s])

    def out_copy(ci, s):
        return pltpu.make_async_copy(
            buf.at[s], o_hbm.at[pl.ds(row_base + ci * _CHUNK, _CHUNK)],
            out_sem.at[s])

    in_copy(0, 0).start()

    def do_chunk(ci, _):
        s = lax.rem(ci, _NSLOT)
        s_next = lax.rem(ci + 1, _NSLOT)
        in_copy(ci, s).wait()

        @pl.when(jnp.logical_and(ci >= 2, ci + 1 < n_chunks))
        def _():
            out_copy(ci - 2, s_next).wait()

        @pl.when(ci + 1 < n_chunks)
        def _():
            in_copy(ci + 1, s_next).start()

        def ld(r, j):
            return buf[s, r, pl.ds(j * _L, _L)]

        def do_row(r, _):
            def blk_body(b, g):
                def inner(i, accs):
                    return tuple(
                        jnp.maximum(a, ld(r, b * _BLK + i * 4 + k))
                        for k, a in enumerate(accs)
                    )

                a0 = tuple(ld(r, b * _BLK + k) for k in range(4))
                accs = plsc.parallel_loop(1, _BLK // 4, carry=a0)(inner)
                bm = jnp.maximum(jnp.maximum(accs[0], accs[1]),
                                 jnp.maximum(accs[2], accs[3]))
                blks[pl.ds(b * _L, _L)] = bm
                return jnp.maximum(g, bm)

            g = lax.fori_loop(0, _NBLK, blk_body, ld(r, 0))

            srt, _ = plsc.sort_key_val(g, g, descending=True)
            l = [srt[i] for i in range(8)]
            m_s = l[0]
            lb = m_s - 1.0
            s_run = l[0] + l[1]
            lb = jnp.maximum(lb, (s_run - 1.0) * 0.5)
            s_run = s_run + l[2]
            lb = jnp.maximum(lb, (s_run - 1.0) * (1.0 / 3.0))
            s_run = s_run + l[3]
            lb = jnp.maximum(lb, (s_run - 1.0) * 0.25)
            s_run = s_run + l[4] + l[5]
            lb = jnp.maximum(lb, (s_run - 1.0) * (1.0 / 6.0))
            s_run = s_run + l[6] + l[7]
            lb = jnp.maximum(lb, (s_run - 1.0) * 0.125)
            m = jnp.broadcast_to(m_s, (_L,))
            thr = jnp.broadcast_to(lb, (_L,))

            cnt = jnp.int32(0)
            for b in range(_NBLK):
                has = plsc.all_reduce_population_count(
                    blks[pl.ds(b * _L, _L)] > thr)[0]

                def scan_block(c, b=b):
                    def sb(j, c):
                        v = ld(r, b * _BLK + j)
                        msk = v > thr
                        plsc.store_compressed(
                            cand.at[pl.ds(c, _L)], v, mask=msk)
                        return c + plsc.all_reduce_population_count(msk)[0]

                    return lax.fori_loop(0, _BLK, sb, c)

                cnt = lax.cond(has > 0, scan_block, lambda c: c, cnt)

            cand[pl.ds(cnt, _L)] = thr - ones
            nvc = (cnt + _L - 1) // _L

            def solve_reg(_):
                v0 = cand[pl.ds(0, _L)]
                sp = [jnp.broadcast_to(v0[i], (_L,)) for i in range(_L)]

                def bis(i, lohi):
                    lo, hi = lohi
                    mid = 0.5 * (lo + hi)
                    s_ = _tree([jnp.maximum(v - mid, 0.0) for v in sp], jnp.add)
                    p = s_ >= ones
                    return jnp.where(p, mid, lo), jnp.where(p, hi, mid)

                lo, hi = lax.fori_loop(0, _N_BISECT, bis, (thr, m))
                tau = 0.5 * (lo + hi)

                def newton(i, tau):
                    ds_ = [v - tau for v in sp]
                    s_ = _tree([jnp.maximum(d, 0.0) for d in ds_], jnp.add)
                    c_ = _tree([jnp.where(d > zeros, 1.0, 0.0) for d in ds_],
                               jnp.add)
                    return tau + (s_ - ones) / jnp.maximum(c_, ones)

                return lax.fori_loop(0, _N_NEWTON, newton, tau)

            def solve_loop(_):
                def bis(i, lohi):
                    lo, hi = lohi
                    mid = 0.5 * (lo + hi)

                    def inner(k, a):
                        v = cand[pl.ds(k * _L, _L)]
                        return a + jnp.maximum(v - mid, 0.0)

                    a = lax.fori_loop(0, nvc, inner, zeros)
                    p = _butterfly(a, jnp.add) >= ones
                    return jnp.where(p, mid, lo), jnp.where(p, hi, mid)

                lo, hi = lax.fori_loop(0, _N_BISECT, bis, (thr, m))
                tau = 0.5 * (lo + hi)

                def newton(i, tau):
                    def inner(k, carry):
                        sa, ca = carry
                        v = cand[pl.ds(k * _L, _L)]
                        d = v - tau
                        sa = sa + jnp.maximum(d, 0.0)
                        ca = ca + jnp.where(d > zeros, 1.0, 0.0)
                        return sa, ca

                    sa, ca = lax.fori_loop(0, nvc, inner, (zeros, zeros))
                    s_ = _butterfly(sa, jnp.add)
                    c_ = _butterfly(ca, jnp.add)
                    return tau + (s_ - ones) / jnp.maximum(c_, ones)

                return lax.fori_loop(0, _N_NEWTON, newton, tau)

            tau = lax.cond(cnt <= _L, solve_reg, solve_loop, 0)

            def out_body(i):
                for k in range(8):
                    sl = pl.ds((i * 8 + k) * _L, _L)
                    buf[s, r, sl] = jnp.maximum(buf[s, r, sl] - tau, 0.0)

            plsc.parallel_loop(0, _NV // 8)(out_body)
            return 0

        lax.fori_loop(0, _CHUNK, do_row, 0)
        out_copy(ci, s).start()
        return 0

    lax.fori_loop(0, n_chunks, do_chunk, 0)
    for ci in (n_chunks - 3, n_chunks - 2, n_chunks - 1):
        out_copy(ci, ci % _NSLOT).wait()


def _sparsemax_sc(x):
    mesh = plsc.VectorSubcoreMesh(core_axis_name="c", subcore_axis_name="s")
    f = pl.kernel(
        _sc_body,
        out_type=jax.ShapeDtypeStruct((_ROWS, _COLS), jnp.float32),
        mesh=mesh,
        scratch_types=[
            pltpu.VMEM((_NSLOT, _CHUNK, _COLS), jnp.float32),
            pltpu.VMEM((_COLS + _L,), jnp.float32),
            pltpu.VMEM((_NBLK * _L,), jnp.float32),
            pltpu.SemaphoreType.DMA((_NSLOT,)),
            pltpu.SemaphoreType.DMA((_NSLOT,)),
        ],
        compiler_params=pltpu.CompilerParams(needs_layout_passes=False),
    )
    return f(x)


def kernel(input):
    return _sparsemax_sc(input)


# speedup vs baseline: 2.6255x; 1.5802x over previous
import functools

import jax
import jax.numpy as jnp
from jax import lax
from jax.experimental import pallas as pl
from jax.experimental.pallas import tpu as pltpu
from jax.experimental.pallas import tpu_sc as plsc

_ROWS = 8192
_COLS = 4096
_L = 16
_NV = _COLS // _L
_BLK = 16
_NBLK = _NV // _BLK
_CHUNK = 8
_NSLOT = 3
_N_WORKERS = 32
_N_BISECT = 12
_N_NEWTON = 2

_GATHER_DNUMS = lax.GatherDimensionNumbers(
    offset_dims=(), collapsed_slice_dims=(0,), start_index_map=(0,)
)


def _shuffle(v, idx):
    return lax.gather(v, idx[:, None], _GATHER_DNUMS, (1,),
                      mode=lax.GatherScatterMode.PROMISE_IN_BOUNDS)


def _butterfly(v, op):
    iota = lax.iota(jnp.int32, _L)
    for k in (8, 4, 2, 1):
        v = op(v, _shuffle(v, jnp.bitwise_xor(iota, k)))
    return v


def _tree(vals, op):
    vals = list(vals)
    while len(vals) > 1:
        nxt = [op(vals[i], vals[i + 1]) for i in range(0, len(vals) - 1, 2)]
        if len(vals) % 2:
            nxt.append(vals[-1])
        vals = nxt
    return vals[0]


def _sc_body(x_hbm, o_hbm, buf, cand, blks, in_sem, out_sem):
    wid = lax.axis_index("s") * 2 + lax.axis_index("c")
    rows_per_w = _ROWS // _N_WORKERS
    n_chunks = rows_per_w // _CHUNK
    row_base = wid * rows_per_w
    ones = jnp.full((_L,), 1.0, jnp.float32)
    zeros = jnp.zeros((_L,), jnp.float32)

    def in_copy(ci, s):
        return pltpu.make_async_copy(
            x_hbm.at[pl.ds(row_base + ci * _CHUNK, _CHUNK)], buf.at[s],
            in_sem.at[s])

    def out_copy(ci, s):
        return pltpu.make_async_copy(
            buf.at[s], o_hbm.at[pl.ds(row_base + ci * _CHUNK, _CHUNK)],
            out_sem.at[s])

    in_copy(0, 0).start()

    def do_chunk(ci, _):
        s = lax.rem(ci, _NSLOT)
        s_next = lax.rem(ci + 1, _NSLOT)
        in_copy(ci, s).wait()

        @pl.when(jnp.logical_and(ci >= 2, ci + 1 < n_chunks))
        def _():
            out_copy(ci - 2, s_next).wait()

        @pl.when(ci + 1 < n_chunks)
        def _():
            in_copy(ci + 1, s_next).start()

        def ld(r, j):
            return buf[s, r, pl.ds(j * _L, _L)]

        def do_row(r, _):
            def blk_body(b, g):
                def inner(i, accs):
                    return tuple(
                        jnp.maximum(a, ld(r, b * _BLK + i * 4 + k))
                        for k, a in enumerate(accs)
                    )

                a0 = tuple(ld(r, b * _BLK + k) for k in range(4))
                accs = plsc.parallel_loop(1, _BLK // 4, carry=a0)(inner)
                bm = jnp.maximum(jnp.maximum(accs[0], accs[1]),
                                 jnp.maximum(accs[2], accs[3]))
                blks[pl.ds(b * _L, _L)] = bm
                return jnp.maximum(g, bm)

            g = lax.fori_loop(0, _NBLK, blk_body, ld(r, 0))

            srt, _ = plsc.sort_key_val(g, g, descending=True)
            l = [srt[i] for i in range(8)]
            m_s = l[0]
            lb = m_s - 1.0
            s_run = l[0] + l[1]
            lb = jnp.maximum(lb, (s_run - 1.0) * 0.5)
            s_run = s_run + l[2]
            lb = jnp.maximum(lb, (s_run - 1.0) * (1.0 / 3.0))
            s_run = s_run + l[3]
            lb = jnp.maximum(lb, (s_run - 1.0) * 0.25)
            s_run = s_run + l[4] + l[5]
            lb = jnp.maximum(lb, (s_run - 1.0) * (1.0 / 6.0))
            s_run = s_run + l[6] + l[7]
            lb = jnp.maximum(lb, (s_run - 1.0) * 0.125)
            m = jnp.broadcast_to(m_s, (_L,))
            thr = jnp.broadcast_to(lb, (_L,))

            cnt = jnp.int32(0)
            for b in range(_NBLK):
                has = plsc.all_reduce_population_count(
                    blks[pl.ds(b * _L, _L)] > thr)[0]

                def scan_block(c, b=b):
                    vs = [ld(r, b * _BLK + j) for j in range(_BLK)]
                    msks = [v > thr for v in vs]
                    pcs = [plsc.all_reduce_population_count(k)[0] for k in msks]
                    offs = [c]
                    for j in range(_BLK - 1):
                        offs.append(offs[-1] + pcs[j])
                    for j in range(_BLK):
                        plsc.store_compressed(
                            cand.at[pl.ds(offs[j], _L)], vs[j], mask=msks[j])
                    return offs[-1] + pcs[-1]

                cnt = lax.cond(has > 0, scan_block, lambda c: c, cnt)

            cand[pl.ds(cnt, _L)] = thr - ones
            nvc = (cnt + _L - 1) // _L

            def solve_reg(_):
                v0 = cand[pl.ds(0, _L)]
                sp = [jnp.broadcast_to(v0[i], (_L,)) for i in range(_L)]

                def bis(i, lohi):
                    lo, hi = lohi
                    mid = 0.5 * (lo + hi)
                    s_ = _tree([jnp.maximum(v - mid, 0.0) for v in sp], jnp.add)
                    p = s_ >= ones
                    return jnp.where(p, mid, lo), jnp.where(p, hi, mid)

                lo, hi = lax.fori_loop(0, _N_BISECT, bis, (thr, m))
                tau = 0.5 * (lo + hi)

                def newton(i, tau):
                    ds_ = [v - tau for v in sp]
                    s_ = _tree([jnp.maximum(d, 0.0) for d in ds_], jnp.add)
                    c_ = _tree([jnp.where(d > zeros, 1.0, 0.0) for d in ds_],
                               jnp.add)
                    return tau + (s_ - ones) / jnp.maximum(c_, ones)

                return lax.fori_loop(0, _N_NEWTON, newton, tau)

            def solve_loop(_):
                def bis(i, lohi):
                    lo, hi = lohi
                    mid = 0.5 * (lo + hi)

                    def inner(k, a):
                        v = cand[pl.ds(k * _L, _L)]
                        return a + jnp.maximum(v - mid, 0.0)

                    a = lax.fori_loop(0, nvc, inner, zeros)
                    p = _butterfly(a, jnp.add) >= ones
                    return jnp.where(p, mid, lo), jnp.where(p, hi, mid)

                lo, hi = lax.fori_loop(0, _N_BISECT, bis, (thr, m))
                tau = 0.5 * (lo + hi)

                def newton(i, tau):
                    def inner(k, carry):
                        sa, ca = carry
                        v = cand[pl.ds(k * _L, _L)]
                        d = v - tau
                        sa = sa + jnp.maximum(d, 0.0)
                        ca = ca + jnp.where(d > zeros, 1.0, 0.0)
                        return sa, ca

                    sa, ca = lax.fori_loop(0, nvc, inner, (zeros, zeros))
                    s_ = _butterfly(sa, jnp.add)
                    c_ = _butterfly(ca, jnp.add)
                    return tau + (s_ - ones) / jnp.maximum(c_, ones)

                return lax.fori_loop(0, _N_NEWTON, newton, tau)

            tau = lax.cond(cnt <= _L, solve_reg, solve_loop, 0)

            def out_body(i):
                for k in range(8):
                    sl = pl.ds((i * 8 + k) * _L, _L)
                    buf[s, r, sl] = jnp.maximum(buf[s, r, sl] - tau, 0.0)

            plsc.parallel_loop(0, _NV // 8)(out_body)
            return 0

        lax.fori_loop(0, _CHUNK, do_row, 0)
        out_copy(ci, s).start()
        return 0

    lax.fori_loop(0, n_chunks, do_chunk, 0)
    for ci in (n_chunks - 3, n_chunks - 2, n_chunks - 1):
        out_copy(ci, ci % _NSLOT).wait()


def _sparsemax_sc(x):
    mesh = plsc.VectorSubcoreMesh(core_axis_name="c", subcore_axis_name="s")
    f = pl.kernel(
        _sc_body,
        out_type=jax.ShapeDtypeStruct((_ROWS, _COLS), jnp.float32),
        mesh=mesh,
        scratch_types=[
            pltpu.VMEM((_NSLOT, _CHUNK, _COLS), jnp.float32),
            pltpu.VMEM((_COLS + _L,), jnp.float32),
            pltpu.VMEM((_NBLK * _L,), jnp.float32),
            pltpu.SemaphoreType.DMA((_NSLOT,)),
            pltpu.SemaphoreType.DMA((_NSLOT,)),
        ],
        compiler_params=pltpu.CompilerParams(needs_layout_passes=False),
    )
    return f(x)


def kernel(input):
    return _sparsemax_sc(input)
